# Initial kernel scaffold; baseline (speedup 1.0000x reference)
import jax
import jax.numpy as jnp
from jax.experimental import pallas as pl


def kernel(x, edge_index, W, Wb, att, bias):
    raise NotImplementedError("write your pallas kernel here")


# speedup vs baseline: 21.6764x; 21.6764x over previous
import functools

import jax
import jax.numpy as jnp
from jax import lax
from jax.experimental import pallas as pl
from jax.experimental.pallas import tpu as pltpu
from jax.experimental.pallas import tpu_sc as plsc

_N = 10000
_E = 320000
_D = 128
_H = 4
_C = 32
_B = 80
_NC = 1
_NS = 16
_NW = _NC * _NS
_EPW = _E // _NW
_NB = _EPW // _B
_NP = 10240
_RPS = _NP // _NS
_DR = 384
_DRS = _DR // _NS
_ROWBLK = 1000
_NEG = 0.2


def _proj_body(x_ref, w_ref, wb_ref, a_ref, h_ref, s_ref, hg_ref, sp_ref):
    h = jnp.dot(x_ref[...], w_ref[...], preferred_element_type=jnp.float32)
    h = h + wb_ref[...]
    h_ref[...] = h
    s = jnp.dot(h, a_ref[...], preferred_element_type=jnp.float32)
    s_ref[...] = s
    pad = jnp.zeros((h.shape[0], _D - 2 * _H), jnp.float32)
    hg_ref[...] = jnp.concatenate([h, s, pad], axis=1)
    sp_ref[...] = jnp.concatenate([s, pad], axis=1)


def _combine_body(a0_ref, d0_ref, h_ref, s_ref, b_ref, r_ref, o_ref):
    s = s_ref[...]
    aa = s[:, 0:_H] + s[:, _H:2 * _H]
    aa = jnp.where(aa >= 0, aa, _NEG * aa)
    wself = jnp.exp(aa)
    rm = r_ref[...]
    wexp = jnp.dot(wself, rm, preferred_element_type=jnp.float32)
    num = a0_ref[...] + wexp * h_ref[...]
    den = d0_ref[...] + wself
    inv = 1.0 / (den + 1e-16)
    invexp = jnp.dot(inv, rm, preferred_element_type=jnp.float32)
    o_ref[...] = jnp.maximum(num * invexp + b_ref[...], 0.0)


def _sc_body(dst_hbm, src_hbm, hg_hbm, sp_hbm, z_hbm, num_hbm, den_hbm,
             dst_v, src_v, drow_v, rows_v, wrow_v, w_v, aid_v,
             acc_sh, den_sh, sem, sem2):
    s = lax.axis_index("s")
    wid = s
    pltpu.sync_copy(z_hbm, acc_sh.at[pl.ds(s * _RPS, _RPS)])
    pltpu.sync_copy(z_hbm.at[pl.ds(0, _DRS)], den_sh.at[pl.ds(s * _DRS, _DRS)])
    plsc.subcore_barrier()

    idx16 = lax.iota(jnp.int32, 16)
    zero16 = jnp.zeros((16,), jnp.float32)

    def batch(b, carry):
        base = wid * _EPW + b * _B
        pltpu.sync_copy(dst_hbm.at[pl.ds(base, _B)], dst_v)
        pltpu.sync_copy(src_hbm.at[pl.ds(base, _B)], src_v)
        cp1 = pltpu.async_copy(hg_hbm.at[src_v], rows_v, sem)
        cp2 = pltpu.async_copy(sp_hbm.at[dst_v], aid_v, sem2)
        cp1.wait()
        cp2.wait()

        def wipe(r, carry2):
            for k in range(_D // 16):
                wrow_v[r, pl.ds(k * 16, 16)] = zero16
            return carry2

        lax.fori_loop(0, _B, wipe, 0)

        for g in range(_B // 16):
            d16 = dst_v[pl.ds(g * 16, 16)]
            s16 = src_v[pl.ds(g * 16, 16)]
            valid = d16 != s16
            col0 = (d16 & 31) * _H
            row16 = idx16 + g * 16
            for hh in range(_H):
                hh16 = jnp.full((16,), hh, jnp.int32)
                ai = plsc.load_gather(aid_v, [row16, hh16])
                aj = plsc.load_gather(rows_v, [row16, hh16 + _D + _H])
                a = ai + aj
                a = jnp.where(a >= 0, a, _NEG * a)
                w = jnp.where(valid, jnp.exp(a), zero16)
                w_v[pl.ds(hh * _B + g * 16, 16)] = w
                plsc.store_scatter(wrow_v, [row16, col0 + hh], w)
            drow_v[pl.ds(g * 16, 16)] = lax.shift_right_logical(d16, 5)

        def edge(r, carry2):
            r16 = jnp.full((16,), r, jnp.int32)
            wb = [plsc.load_gather(w_v, [r16 + hh * _B]) for hh in range(_H)]
            for k in range(_D // 16):
                aid_v[r, pl.ds(k * 16, 16)] = (
                    rows_v[r, pl.ds(k * 16, 16)] * wb[k // 2])
            return carry2

        lax.fori_loop(0, _B, edge, 0)
        pltpu.sync_copy(aid_v, acc_sh.at[dst_v], add=True)
        pltpu.sync_copy(wrow_v, den_sh.at[drow_v], add=True)
        return carry

    lax.fori_loop(0, _NB, batch, 0)
    plsc.subcore_barrier()
    pltpu.sync_copy(acc_sh.at[pl.ds(s * _RPS, _RPS)],
                    num_hbm.at[pl.ds(s * _RPS, _RPS)])
    pltpu.sync_copy(den_sh.at[pl.ds(s * _DRS, _DRS)],
                    den_hbm.at[pl.ds(s * _DRS, _DRS)])


_sc_call = functools.partial(
    pl.kernel,
    out_type=[
        jax.ShapeDtypeStruct((_NP, _D), jnp.float32),
        jax.ShapeDtypeStruct((_DR, _D), jnp.float32),
    ],
    compiler_params=pltpu.CompilerParams(needs_layout_passes=False),
    mesh=plsc.VectorSubcoreMesh(core_axis_name="c", subcore_axis_name="s",
                                num_cores=_NC, num_subcores=_NS),
    scratch_types=[
        pltpu.VMEM((_B,), jnp.int32),
        pltpu.VMEM((_B,), jnp.int32),
        pltpu.VMEM((_B,), jnp.int32),
        pltpu.VMEM((_B, 2 * _D), jnp.float32),
        pltpu.VMEM((_B, _D), jnp.float32),
        pltpu.VMEM((2 * _H * _B,), jnp.float32),
        pltpu.VMEM((_B, _D), jnp.float32),
        pltpu.VMEM_SHARED((_NP, _D), jnp.float32),
        pltpu.VMEM_SHARED((_DR, _D), jnp.float32),
        pltpu.SemaphoreType.DMA,
        pltpu.SemaphoreType.DMA,
    ],
)(_sc_body)


def kernel(x, edge_index, W, Wb, att, bias):
    m = jnp.repeat(jnp.eye(_H, dtype=jnp.float32), _C, axis=0)
    a_dst = att[0, :, :_C].reshape(_D)
    a_src = att[0, :, _C:].reshape(_D)
    amat = jnp.concatenate([a_dst[:, None] * m, a_src[:, None] * m], axis=1)

    h, scal, haug, scalpad = pl.pallas_call(
        _proj_body,
        grid=(_N // _ROWBLK,),
        in_specs=[
            pl.BlockSpec((_ROWBLK, _D), lambda i: (i, 0)),
            pl.BlockSpec((_D, _D), lambda i: (0, 0)),
            pl.BlockSpec((1, _D), lambda i: (0, 0)),
            pl.BlockSpec((_D, 2 * _H), lambda i: (0, 0)),
        ],
        out_specs=[
            pl.BlockSpec((_ROWBLK, _D), lambda i: (i, 0)),
            pl.BlockSpec((_ROWBLK, 2 * _H), lambda i: (i, 0)),
            pl.BlockSpec((_ROWBLK, 2 * _D), lambda i: (i, 0)),
            pl.BlockSpec((_ROWBLK, _D), lambda i: (i, 0)),
        ],
        out_shape=[
            jax.ShapeDtypeStruct((_N, _D), jnp.float32),
            jax.ShapeDtypeStruct((_N, 2 * _H), jnp.float32),
            jax.ShapeDtypeStruct((_N, 2 * _D), jnp.float32),
            jax.ShapeDtypeStruct((_N, _D), jnp.float32),
        ],
    )(x, W, Wb.reshape(1, _D), amat)

    zeros = jnp.zeros((_RPS, _D), jnp.float32)
    num, den = _sc_call(edge_index[0], edge_index[1], haug, scalpad, zeros)
    den = den.reshape(_DR * _D)[:_N * _H].reshape(_N, _H)

    out = pl.pallas_call(
        _combine_body,
        grid=(_N // _ROWBLK,),
        in_specs=[
            pl.BlockSpec((_ROWBLK, _D), lambda i: (i, 0)),
            pl.BlockSpec((_ROWBLK, _H), lambda i: (i, 0)),
            pl.BlockSpec((_ROWBLK, _D), lambda i: (i, 0)),
            pl.BlockSpec((_ROWBLK, 2 * _H), lambda i: (i, 0)),
            pl.BlockSpec((1, _D), lambda i: (0, 0)),
            pl.BlockSpec((_H, _D), lambda i: (0, 0)),
        ],
        out_specs=pl.BlockSpec((_ROWBLK, _D), lambda i: (i, 0)),
        out_shape=jax.ShapeDtypeStruct((_N, _D), jnp.float32),
    )(num[:_N], den, h, scal, bias.reshape(1, _D), m.T)
    return out


# speedup vs baseline: 40.5362x; 1.8701x over previous
import functools

import jax
import jax.numpy as jnp
from jax import lax
from jax.experimental import pallas as pl
from jax.experimental.pallas import tpu as pltpu
from jax.experimental.pallas import tpu_sc as plsc

_N = 10000
_E = 320000
_D = 128
_H = 4
_C = 32
_B = 80
_NC = 2
_NS = 16
_NW = _NC * _NS
_EPW = _E // _NW
_NB = _EPW // _B
_NP = 10240
_RPS = _NP // _NS
_DR = 384
_DRS = _DR // _NS
_ROWBLK = 1000
_NEG = 0.2


def _proj_body(x_ref, w_ref, wb_ref, a_ref, h_ref, s_ref, hg_ref, sp_ref):
    h = jnp.dot(x_ref[...], w_ref[...], preferred_element_type=jnp.float32)
    h = h + wb_ref[...]
    h_ref[...] = h
    s = jnp.dot(h, a_ref[...], preferred_element_type=jnp.float32)
    s_ref[...] = s
    pad = jnp.zeros((h.shape[0], _D - 2 * _H), jnp.float32)
    hg_ref[...] = jnp.concatenate([h, s, pad], axis=1)
    sp_ref[...] = jnp.concatenate([s, pad], axis=1)


def _combine_body(a0_ref, a1_ref, d0_ref, d1_ref, h_ref, s_ref, b_ref,
                  r_ref, o_ref):
    s = s_ref[...]
    aa = s[:, 0:_H] + s[:, _H:2 * _H]
    aa = jnp.where(aa >= 0, aa, _NEG * aa)
    wself = jnp.exp(aa)
    rm = r_ref[...]
    wexp = jnp.dot(wself, rm, preferred_element_type=jnp.float32)
    num = a0_ref[...] + a1_ref[...] + wexp * h_ref[...]
    den = d0_ref[...] + d1_ref[...] + wself
    inv = 1.0 / (den + 1e-16)
    invexp = jnp.dot(inv, rm, preferred_element_type=jnp.float32)
    o_ref[...] = jnp.maximum(num * invexp + b_ref[...], 0.0)


def _sc_body(dst_hbm, src_hbm, hg_hbm, sp_hbm, z_hbm, num_hbm, den_hbm,
             dst_v, src_v, drow_v, rows_v, wrow_v, w_v, aid_v,
             acc_sh, den_sh, sem, sem2):
    c = lax.axis_index("c")
    s = lax.axis_index("s")
    wid = s * _NC + c
    pltpu.sync_copy(z_hbm, acc_sh.at[pl.ds(s * _RPS, _RPS)])
    pltpu.sync_copy(z_hbm.at[pl.ds(0, _DRS)], den_sh.at[pl.ds(s * _DRS, _DRS)])
    plsc.subcore_barrier()

    idx16 = lax.iota(jnp.int32, 16)
    zero16 = jnp.zeros((16,), jnp.float32)

    def batch(b, carry):
        base = wid * _EPW + b * _B
        pltpu.sync_copy(dst_hbm.at[pl.ds(base, _B)], dst_v)
        pltpu.sync_copy(src_hbm.at[pl.ds(base, _B)], src_v)
        cp1 = pltpu.async_copy(hg_hbm.at[src_v], rows_v, sem)
        cp2 = pltpu.async_copy(sp_hbm.at[dst_v], aid_v, sem2)
        cp1.wait()
        cp2.wait()

        def wipe(r, carry2):
            for k in range(_D // 16):
                wrow_v[r, pl.ds(k * 16, 16)] = zero16
            return carry2

        lax.fori_loop(0, _B, wipe, 0)

        for g in range(_B // 16):
            d16 = dst_v[pl.ds(g * 16, 16)]
            s16 = src_v[pl.ds(g * 16, 16)]
            valid = d16 != s16
            col0 = (d16 & 31) * _H
            row16 = idx16 + g * 16
            for hh in range(_H):
                hh16 = jnp.full((16,), hh, jnp.int32)
                ai = plsc.load_gather(aid_v, [row16, hh16])
                aj = plsc.load_gather(rows_v, [row16, hh16 + _D + _H])
                a = ai + aj
                a = jnp.where(a >= 0, a, _NEG * a)
                w = jnp.where(valid, jnp.exp(a), zero16)
                w_v[pl.ds(hh * _B + g * 16, 16)] = w
                plsc.store_scatter(wrow_v, [row16, col0 + hh], w)
            drow_v[pl.ds(g * 16, 16)] = lax.shift_right_logical(d16, 5)

        def edge(r, carry2):
            r16 = jnp.full((16,), r, jnp.int32)
            wb = [plsc.load_gather(w_v, [r16 + hh * _B]) for hh in range(_H)]
            for k in range(_D // 16):
                aid_v[r, pl.ds(k * 16, 16)] = (
                    rows_v[r, pl.ds(k * 16, 16)] * wb[k // 2])
            return carry2

        lax.fori_loop(0, _B, edge, 0)
        pltpu.sync_copy(aid_v, acc_sh.at[dst_v], add=True)
        pltpu.sync_copy(wrow_v, den_sh.at[drow_v], add=True)
        return carry

    lax.fori_loop(0, _NB, batch, 0)
    plsc.subcore_barrier()
    pltpu.sync_copy(acc_sh.at[pl.ds(s * _RPS, _RPS)],
                    num_hbm.at[c, pl.ds(s * _RPS, _RPS)])
    pltpu.sync_copy(den_sh.at[pl.ds(s * _DRS, _DRS)],
                    den_hbm.at[c, pl.ds(s * _DRS, _DRS)])


_sc_call = functools.partial(
    pl.kernel,
    out_type=[
        jax.ShapeDtypeStruct((_NC, _NP, _D), jnp.float32),
        jax.ShapeDtypeStruct((_NC, _DR, _D), jnp.float32),
    ],
    compiler_params=pltpu.CompilerParams(needs_layout_passes=False),
    mesh=plsc.VectorSubcoreMesh(core_axis_name="c", subcore_axis_name="s",
                                num_cores=_NC, num_subcores=_NS),
    scratch_types=[
        pltpu.VMEM((_B,), jnp.int32),
        pltpu.VMEM((_B,), jnp.int32),
        pltpu.VMEM((_B,), jnp.int32),
        pltpu.VMEM((_B, 2 * _D), jnp.float32),
        pltpu.VMEM((_B, _D), jnp.float32),
        pltpu.VMEM((2 * _H * _B,), jnp.float32),
        pltpu.VMEM((_B, _D), jnp.float32),
        pltpu.VMEM_SHARED((_NP, _D), jnp.float32),
        pltpu.VMEM_SHARED((_DR, _D), jnp.float32),
        pltpu.SemaphoreType.DMA,
        pltpu.SemaphoreType.DMA,
    ],
)(_sc_body)


def kernel(x, edge_index, W, Wb, att, bias):
    m = jnp.repeat(jnp.eye(_H, dtype=jnp.float32), _C, axis=0)
    a_dst = att[0, :, :_C].reshape(_D)
    a_src = att[0, :, _C:].reshape(_D)
    amat = jnp.concatenate([a_dst[:, None] * m, a_src[:, None] * m], axis=1)

    h, scal, haug, scalpad = pl.pallas_call(
        _proj_body,
        grid=(_N // _ROWBLK,),
        in_specs=[
            pl.BlockSpec((_ROWBLK, _D), lambda i: (i, 0)),
            pl.BlockSpec((_D, _D), lambda i: (0, 0)),
            pl.BlockSpec((1, _D), lambda i: (0, 0)),
            pl.BlockSpec((_D, 2 * _H), lambda i: (0, 0)),
        ],
        out_specs=[
            pl.BlockSpec((_ROWBLK, _D), lambda i: (i, 0)),
            pl.BlockSpec((_ROWBLK, 2 * _H), lambda i: (i, 0)),
            pl.BlockSpec((_ROWBLK, 2 * _D), lambda i: (i, 0)),
            pl.BlockSpec((_ROWBLK, _D), lambda i: (i, 0)),
        ],
        out_shape=[
            jax.ShapeDtypeStruct((_N, _D), jnp.float32),
            jax.ShapeDtypeStruct((_N, 2 * _H), jnp.float32),
            jax.ShapeDtypeStruct((_N, 2 * _D), jnp.float32),
            jax.ShapeDtypeStruct((_N, _D), jnp.float32),
        ],
    )(x, W, Wb.reshape(1, _D), amat)

    zeros = jnp.zeros((_RPS, _D), jnp.float32)
    num, den = _sc_call(edge_index[0], edge_index[1], haug, scalpad, zeros)
    den = den.reshape(_NC, _DR * _D)[:, :_N * _H].reshape(_NC, _N, _H)

    out = pl.pallas_call(
        _combine_body,
        grid=(_N // _ROWBLK,),
        in_specs=[
            pl.BlockSpec((_ROWBLK, _D), lambda i: (i, 0)),
            pl.BlockSpec((_ROWBLK, _D), lambda i: (i, 0)),
            pl.BlockSpec((_ROWBLK, _H), lambda i: (i, 0)),
            pl.BlockSpec((_ROWBLK, _H), lambda i: (i, 0)),
            pl.BlockSpec((_ROWBLK, _D), lambda i: (i, 0)),
            pl.BlockSpec((_ROWBLK, 2 * _H), lambda i: (i, 0)),
            pl.BlockSpec((1, _D), lambda i: (0, 0)),
            pl.BlockSpec((_H, _D), lambda i: (0, 0)),
        ],
        out_specs=pl.BlockSpec((_ROWBLK, _D), lambda i: (i, 0)),
        out_shape=jax.ShapeDtypeStruct((_N, _D), jnp.float32),
    )(num[0, :_N], num[1, :_N], den[0], den[1], h, scal,
      bias.reshape(1, _D), m.T)
    return out


# speedup vs baseline: 47.1449x; 1.1630x over previous
import functools

import jax
import jax.numpy as jnp
from jax import lax
from jax.experimental import pallas as pl
from jax.experimental.pallas import tpu as pltpu
from jax.experimental.pallas import tpu_sc as plsc

_N = 10000
_E = 320000
_D = 128
_H = 4
_C = 32
_B = 80
_NC = 2
_NS = 16
_NW = _NC * _NS
_EPW = _E // _NW
_NB = _EPW // _B
_NP = 10240
_RPS = _NP // _NS
_DR = 384
_DRS = _DR // _NS
_ROWBLK = 1000
_NEG = 0.2


def _proj_body(x_ref, w_ref, wb_ref, a_ref, h_ref, s_ref, hg_ref, sp_ref):
    h = jnp.dot(x_ref[...], w_ref[...], preferred_element_type=jnp.float32)
    h = h + wb_ref[...]
    h_ref[...] = h
    s = jnp.dot(h, a_ref[...], preferred_element_type=jnp.float32)
    s_ref[...] = s
    pad = jnp.zeros((h.shape[0], _D - 2 * _H), jnp.float32)
    hg_ref[...] = jnp.concatenate([h, s, pad], axis=1)
    sp_ref[...] = jnp.concatenate([s, pad], axis=1)


def _combine_body(a0_ref, a1_ref, d0_ref, d1_ref, h_ref, s_ref, b_ref,
                  r_ref, o_ref):
    s = s_ref[...]
    aa = s[:, 0:_H] + s[:, _H:2 * _H]
    aa = jnp.where(aa >= 0, aa, _NEG * aa)
    wself = jnp.exp(aa)
    rm = r_ref[...]
    wexp = jnp.dot(wself, rm, preferred_element_type=jnp.float32)
    num = a0_ref[...] + a1_ref[...] + wexp * h_ref[...]
    den = d0_ref[...] + d1_ref[...] + wself
    inv = 1.0 / (den + 1e-16)
    invexp = jnp.dot(inv, rm, preferred_element_type=jnp.float32)
    o_ref[...] = jnp.maximum(num * invexp + b_ref[...], 0.0)


def _sc_body(dst_hbm, src_hbm, hg_hbm, sp_hbm, z_hbm, num_hbm, den_hbm,
             dst_v, src_v, dsc_v, drow_v, rows_v, wrow_v, w_v, aid_v,
             acc_sh, den_sh, sem, sem2, sem4, sem5):
    c = lax.axis_index("c")
    s = lax.axis_index("s")
    wid = s * _NC + c
    pltpu.sync_copy(z_hbm, acc_sh.at[pl.ds(s * _RPS, _RPS)])
    pltpu.sync_copy(z_hbm.at[pl.ds(0, _DRS)], den_sh.at[pl.ds(s * _DRS, _DRS)])
    plsc.subcore_barrier()

    idx16 = lax.iota(jnp.int32, 16)
    zero16 = jnp.zeros((16,), jnp.float32)

    def batch(b, carry):
        base = wid * _EPW + b * _B
        pltpu.sync_copy(dst_hbm.at[pl.ds(base, _B)], dst_v)
        pltpu.sync_copy(src_hbm.at[pl.ds(base, _B)], src_v)
        cp1 = pltpu.async_copy(hg_hbm.at[src_v], rows_v, sem)

        @pl.when(b > 0)
        def _drain():
            pltpu.make_async_copy(aid_v, acc_sh.at[dsc_v], sem4).wait()
            pltpu.make_async_copy(wrow_v, den_sh.at[drow_v], sem5).wait()

        cp2 = pltpu.async_copy(sp_hbm.at[dst_v], aid_v, sem2)

        def wipe(r, carry2):
            for k in range(_D // 16):
                wrow_v[r, pl.ds(k * 16, 16)] = zero16
            return carry2

        lax.fori_loop(0, _B, wipe, 0)
        cp1.wait()
        cp2.wait()

        for g in range(_B // 16):
            d16 = dst_v[pl.ds(g * 16, 16)]
            s16 = src_v[pl.ds(g * 16, 16)]
            valid = d16 != s16
            col0 = (d16 & 31) * _H
            row16 = idx16 + g * 16
            for hh in range(_H):
                hh16 = jnp.full((16,), hh, jnp.int32)
                ai = plsc.load_gather(aid_v, [row16, hh16])
                aj = plsc.load_gather(rows_v, [row16, hh16 + _D + _H])
                a = ai + aj
                a = jnp.where(a >= 0, a, _NEG * a)
                w = jnp.where(valid, jnp.exp(a), zero16)
                w_v[pl.ds(hh * _B + g * 16, 16)] = w
                plsc.store_scatter(wrow_v, [row16, col0 + hh], w)
            drow_v[pl.ds(g * 16, 16)] = lax.shift_right_logical(d16, 5)

        def edge(r, carry2):
            r16 = jnp.full((16,), r, jnp.int32)
            wb = [plsc.load_gather(w_v, [r16 + hh * _B]) for hh in range(_H)]
            for k in range(_D // 16):
                aid_v[r, pl.ds(k * 16, 16)] = (
                    rows_v[r, pl.ds(k * 16, 16)] * wb[k // 2])
            return carry2

        lax.fori_loop(0, _B, edge, 0)
        for t in range(_B // 16):
            dsc_v[pl.ds(t * 16, 16)] = dst_v[pl.ds(t * 16, 16)]
        pltpu.async_copy(aid_v, acc_sh.at[dsc_v], sem4, add=True)
        pltpu.async_copy(wrow_v, den_sh.at[drow_v], sem5, add=True)
        return carry

    lax.fori_loop(0, _NB, batch, 0)
    pltpu.make_async_copy(aid_v, acc_sh.at[dsc_v], sem4).wait()
    pltpu.make_async_copy(wrow_v, den_sh.at[drow_v], sem5).wait()
    plsc.subcore_barrier()
    pltpu.sync_copy(acc_sh.at[pl.ds(s * _RPS, _RPS)],
                    num_hbm.at[c, pl.ds(s * _RPS, _RPS)])
    pltpu.sync_copy(den_sh.at[pl.ds(s * _DRS, _DRS)],
                    den_hbm.at[c, pl.ds(s * _DRS, _DRS)])


_sc_call = functools.partial(
    pl.kernel,
    out_type=[
        jax.ShapeDtypeStruct((_NC, _NP, _D), jnp.float32),
        jax.ShapeDtypeStruct((_NC, _DR, _D), jnp.float32),
    ],
    compiler_params=pltpu.CompilerParams(needs_layout_passes=False),
    mesh=plsc.VectorSubcoreMesh(core_axis_name="c", subcore_axis_name="s",
                                num_cores=_NC, num_subcores=_NS),
    scratch_types=[
        pltpu.VMEM((_B,), jnp.int32),
        pltpu.VMEM((_B,), jnp.int32),
        pltpu.VMEM((_B,), jnp.int32),
        pltpu.VMEM((_B,), jnp.int32),
        pltpu.VMEM((_B, 2 * _D), jnp.float32),
        pltpu.VMEM((_B, _D), jnp.float32),
        pltpu.VMEM((2 * _H * _B,), jnp.float32),
        pltpu.VMEM((_B, _D), jnp.float32),
        pltpu.VMEM_SHARED((_NP, _D), jnp.float32),
        pltpu.VMEM_SHARED((_DR, _D), jnp.float32),
        pltpu.SemaphoreType.DMA,
        pltpu.SemaphoreType.DMA,
        pltpu.SemaphoreType.DMA,
        pltpu.SemaphoreType.DMA,
    ],
)(_sc_body)


def kernel(x, edge_index, W, Wb, att, bias):
    m = jnp.repeat(jnp.eye(_H, dtype=jnp.float32), _C, axis=0)
    a_dst = att[0, :, :_C].reshape(_D)
    a_src = att[0, :, _C:].reshape(_D)
    amat = jnp.concatenate([a_dst[:, None] * m, a_src[:, None] * m], axis=1)

    h, scal, haug, scalpad = pl.pallas_call(
        _proj_body,
        grid=(_N // _ROWBLK,),
        in_specs=[
            pl.BlockSpec((_ROWBLK, _D), lambda i: (i, 0)),
            pl.BlockSpec((_D, _D), lambda i: (0, 0)),
            pl.BlockSpec((1, _D), lambda i: (0, 0)),
            pl.BlockSpec((_D, 2 * _H), lambda i: (0, 0)),
        ],
        out_specs=[
            pl.BlockSpec((_ROWBLK, _D), lambda i: (i, 0)),
            pl.BlockSpec((_ROWBLK, 2 * _H), lambda i: (i, 0)),
            pl.BlockSpec((_ROWBLK, 2 * _D), lambda i: (i, 0)),
            pl.BlockSpec((_ROWBLK, _D), lambda i: (i, 0)),
        ],
        out_shape=[
            jax.ShapeDtypeStruct((_N, _D), jnp.float32),
            jax.ShapeDtypeStruct((_N, 2 * _H), jnp.float32),
            jax.ShapeDtypeStruct((_N, 2 * _D), jnp.float32),
            jax.ShapeDtypeStruct((_N, _D), jnp.float32),
        ],
    )(x, W, Wb.reshape(1, _D), amat)

    zeros = jnp.zeros((_RPS, _D), jnp.float32)
    num, den = _sc_call(edge_index[0], edge_index[1], haug, scalpad, zeros)
    den = den.reshape(_NC, _DR * _D)[:, :_N * _H].reshape(_NC, _N, _H)

    out = pl.pallas_call(
        _combine_body,
        grid=(_N // _ROWBLK,),
        in_specs=[
            pl.BlockSpec((_ROWBLK, _D), lambda i: (i, 0)),
            pl.BlockSpec((_ROWBLK, _D), lambda i: (i, 0)),
            pl.BlockSpec((_ROWBLK, _H), lambda i: (i, 0)),
            pl.BlockSpec((_ROWBLK, _H), lambda i: (i, 0)),
            pl.BlockSpec((_ROWBLK, _D), lambda i: (i, 0)),
            pl.BlockSpec((_ROWBLK, 2 * _H), lambda i: (i, 0)),
            pl.BlockSpec((1, _D), lambda i: (0, 0)),
            pl.BlockSpec((_H, _D), lambda i: (0, 0)),
        ],
        out_specs=pl.BlockSpec((_ROWBLK, _D), lambda i: (i, 0)),
        out_shape=jax.ShapeDtypeStruct((_N, _D), jnp.float32),
    )(num[0, :_N], num[1, :_N], den[0], den[1], h, scal,
      bias.reshape(1, _D), m.T)
    return out
